# Initial kernel scaffold; baseline (speedup 1.0000x reference)
import jax
import jax.numpy as jnp
from jax.experimental import pallas as pl


def kernel(x, w1, s1, t1, w2, s2, t2, w3, s3, t3, fw1, fb1, fw2, fb2):
    raise NotImplementedError("write your pallas kernel here")


# speedup vs baseline: 13.3259x; 13.3259x over previous
import numpy as np

import jax
import jax.numpy as jnp
from jax.experimental import pallas as pl
from jax.experimental.pallas import tpu as pltpu

B = 16

_L1 = (5, 3, 16, 32, 36)
_L2 = (5, 16, 32, 16, 20)
_L3 = (3, 32, 64, 8, 10)


def _build_T(w, K, Cin, Cout, Wout, Wpos):
    w4 = w.reshape(K, K, Cin, Cout).astype(jnp.float32)
    j = np.arange(K)[:, None, None]
    wo = np.arange(Wout)[None, :, None]
    wp = np.arange(Wpos)[None, None, :]
    O = (wp == wo + j).astype(np.float32)
    T = jnp.einsum('jwp,ijcd->ipcwd', O, w4)
    return T.reshape(K, Wpos * Cin, Wout * Cout).astype(jnp.bfloat16)


def _build_sel(Wout_pooled, C, halo, in_lanes, out_lanes):
    S = np.zeros((in_lanes, out_lanes), np.float32)
    for w in range(Wout_pooled):
        for c in range(C):
            S[(2 * w) * C + c, (halo + w) * C + c] = 1.0
    return jnp.asarray(S, jnp.bfloat16)


_SEL1 = _build_sel(16, 16, 2, 496, 320)
_SEL2 = _build_sel(8, 32, 1, 480, 320)


def _cnn_body(xt_ref, t1_ref, sc1_ref, sh1_ref, sel1_ref,
              t2_ref, sc2_ref, sh2_ref, sel2_ref,
              t3_ref, sc3_ref, sh3_ref,
              wf1_ref, fb1_ref, fw2_ref, fb2_ref,
              o_ref, a2_ref, a3_ref):
    f32 = jnp.float32

    acc = None
    for i in range(5):
        lhs = xt_ref[:, i:i + 32, :].reshape(B * 32, 108)
        d = jnp.dot(lhs, t1_ref[i], preferred_element_type=f32)
        acc = d if acc is None else acc + d
    y = jnp.maximum(acc * sc1_ref[...] + sh1_ref[...], 0.0)
    v = jnp.max(y.reshape(B, 16, 2, 512), axis=2)
    h = jnp.maximum(v[:, :, :496], v[:, :, 16:])
    p = jnp.dot(h.reshape(B * 16, 496).astype(jnp.bfloat16), sel1_ref[...],
                preferred_element_type=f32)
    a2_ref[:, 0:2, :] = jnp.zeros((B, 2, 320), jnp.bfloat16)
    a2_ref[:, 18:20, :] = jnp.zeros((B, 2, 320), jnp.bfloat16)
    a2_ref[:, 2:18, :] = p.astype(jnp.bfloat16).reshape(B, 16, 320)

    acc = None
    for i in range(5):
        lhs = a2_ref[:, i:i + 16, :].reshape(B * 16, 320)
        d = jnp.dot(lhs, t2_ref[i], preferred_element_type=f32)
        acc = d if acc is None else acc + d
    y = jnp.maximum(acc * sc2_ref[...] + sh2_ref[...], 0.0)
    v = jnp.max(y.reshape(B, 8, 2, 512), axis=2)
    h = jnp.maximum(v[:, :, :480], v[:, :, 32:])
    p = jnp.dot(h.reshape(B * 8, 480).astype(jnp.bfloat16), sel2_ref[...],
                preferred_element_type=f32)
    a3_ref[:, 0:1, :] = jnp.zeros((B, 1, 320), f32)
    a3_ref[:, 9:10, :] = jnp.zeros((B, 1, 320), f32)
    a3_ref[:, 1:9, :] = p.reshape(B, 8, 320)

    acc = None
    for i in range(3):
        lhs = a3_ref[:, i:i + 8, :].reshape(B * 8, 320).astype(jnp.bfloat16)
        d = jnp.dot(lhs, t3_ref[i], preferred_element_type=f32)
        acc = d if acc is None else acc + d
    y = jnp.maximum(acc * sc3_ref[...] + sh3_ref[...], 0.0)
    v = jnp.max(y.reshape(B, 4, 2, 512), axis=2)
    h = jnp.maximum(v[:, :, :448], v[:, :, 64:])

    f = None
    for hh in range(4):
        d = jnp.dot(h[:, hh, :].astype(jnp.bfloat16), wf1_ref[hh],
                    preferred_element_type=f32)
        f = d if f is None else f + d
    hrelu = jnp.maximum(f + fb1_ref[...], 0.0)
    o_ref[...] = jnp.dot(hrelu, fw2_ref[...],
                         preferred_element_type=f32) + fb2_ref[...]


def kernel(x, w1, s1, t1, w2, s2, t2, w3, s3, t3, fw1, fb1, fw2, fb2):
    n = x.shape[0]

    xt = jnp.transpose(x, (0, 2, 3, 1))
    xt = jnp.pad(xt, ((0, 0), (2, 2), (2, 2), (0, 0)))
    xt = xt.reshape(n, 36, 108).astype(jnp.bfloat16)

    T1 = _build_T(w1, *_L1)
    T2 = _build_T(w2, *_L2)
    T3 = _build_T(w3, *_L3)
    sc1, sh1 = jnp.tile(s1, (1, 32)), jnp.tile(t1, (1, 32))
    sc2, sh2 = jnp.tile(s2, (1, 16)), jnp.tile(t2, (1, 16))
    sc3, sh3 = jnp.tile(s3, (1, 8)), jnp.tile(t3, (1, 8))

    f4 = fw1.reshape(4, 4, 64, 256)
    Wf1 = jnp.zeros((4, 7, 64, 256), fw1.dtype).at[:, ::2].set(f4)
    Wf1 = Wf1.reshape(4, 448, 256)

    full = lambda shape: pl.BlockSpec(shape, lambda i: tuple(0 for _ in shape))
    in_specs = [
        pl.BlockSpec((B, 36, 108), lambda i: (i, 0, 0)),
        full((5, 108, 512)), full((1, 512)), full((1, 512)), full((496, 320)),
        full((5, 320, 512)), full((1, 512)), full((1, 512)), full((480, 320)),
        full((3, 320, 512)), full((1, 512)), full((1, 512)),
        full((4, 448, 256)), full((1, 256)), full((256, 2)), full((1, 2)),
    ]
    out = pl.pallas_call(
        _cnn_body,
        out_shape=jax.ShapeDtypeStruct((n, 2), jnp.float32),
        grid=(n // B,),
        in_specs=in_specs,
        out_specs=pl.BlockSpec((B, 2), lambda i: (i, 0)),
        scratch_shapes=[
            pltpu.VMEM((B, 20, 320), jnp.bfloat16),
            pltpu.VMEM((B, 10, 320), jnp.float32),
        ],
        compiler_params=pltpu.CompilerParams(dimension_semantics=("parallel",)),
    )(xt, T1, sc1, sh1, _SEL1, T2, sc2, sh2, _SEL2,
      T3, sc3, sh3, Wf1, fb1, fw2, fb2)
    return out


# speedup vs baseline: 13.3297x; 1.0003x over previous
import numpy as np

import jax
import jax.numpy as jnp
from jax.experimental import pallas as pl
from jax.experimental.pallas import tpu as pltpu

B = 16

_L1 = (5, 3, 16, 32, 36)
_L2 = (5, 16, 32, 16, 20)
_L3 = (3, 32, 64, 8, 10)


def _build_T(w, K, Cin, Cout, Wout, Wpos):
    w4 = w.reshape(K, K, Cin, Cout).astype(jnp.float32)
    j = np.arange(K)[:, None, None]
    wo = np.arange(Wout)[None, :, None]
    wp = np.arange(Wpos)[None, None, :]
    O = (wp == wo + j).astype(np.float32)
    T = jnp.einsum('jwp,ijcd->ipcwd', O, w4)
    return T.reshape(K, Wpos * Cin, Wout * Cout).astype(jnp.bfloat16)


def _build_sel(Wout_pooled, C, halo, in_lanes, out_lanes):
    S = np.zeros((in_lanes, out_lanes), np.float32)
    for w in range(Wout_pooled):
        for c in range(C):
            S[(2 * w) * C + c, (halo + w) * C + c] = 1.0
    return S


_SEL1 = _build_sel(16, 16, 2, 496, 320)
_SEL2 = _build_sel(8, 32, 1, 480, 320)


def _cnn_body(xt_ref, t1_ref, sc1_ref, sh1_ref, sel1_ref,
              t2_ref, sc2_ref, sh2_ref, sel2_ref,
              t3_ref, sc3_ref, sh3_ref,
              wf1_ref, fb1_ref, fw2_ref, fb2_ref,
              o_ref, a2_ref, a3_ref):
    f32 = jnp.float32

    acc = None
    for i in range(5):
        lhs = xt_ref[:, i:i + 32, :].reshape(B * 32, 108)
        d = jnp.dot(lhs, t1_ref[i], preferred_element_type=f32)
        acc = d if acc is None else acc + d
    y = jnp.maximum(acc * sc1_ref[...] + sh1_ref[...], 0.0)
    v = jnp.max(y.reshape(B, 16, 2, 512), axis=2)
    h = jnp.maximum(v[:, :, :496], v[:, :, 16:])
    p = jnp.dot(h.reshape(B * 16, 496).astype(jnp.bfloat16), sel1_ref[...],
                preferred_element_type=f32)
    a2_ref[:, 0:2, :] = jnp.zeros((B, 2, 320), jnp.bfloat16)
    a2_ref[:, 18:20, :] = jnp.zeros((B, 2, 320), jnp.bfloat16)
    a2_ref[:, 2:18, :] = p.astype(jnp.bfloat16).reshape(B, 16, 320)

    acc = None
    for i in range(5):
        lhs = a2_ref[:, i:i + 16, :].reshape(B * 16, 320)
        d = jnp.dot(lhs, t2_ref[i], preferred_element_type=f32)
        acc = d if acc is None else acc + d
    y = jnp.maximum(acc * sc2_ref[...] + sh2_ref[...], 0.0)
    v = jnp.max(y.reshape(B, 8, 2, 512), axis=2)
    h = jnp.maximum(v[:, :, :480], v[:, :, 32:])
    p = jnp.dot(h.reshape(B * 8, 480).astype(jnp.bfloat16), sel2_ref[...],
                preferred_element_type=f32)
    a3_ref[:, 0:1, :] = jnp.zeros((B, 1, 320), f32)
    a3_ref[:, 9:10, :] = jnp.zeros((B, 1, 320), f32)
    a3_ref[:, 1:9, :] = p.reshape(B, 8, 320)

    acc = None
    for i in range(3):
        lhs = a3_ref[:, i:i + 8, :].reshape(B * 8, 320).astype(jnp.bfloat16)
        d = jnp.dot(lhs, t3_ref[i], preferred_element_type=f32)
        acc = d if acc is None else acc + d
    y = jnp.maximum(acc * sc3_ref[...] + sh3_ref[...], 0.0)
    v = jnp.max(y.reshape(B, 4, 2, 512), axis=2)
    h = jnp.maximum(v[:, :, :448], v[:, :, 64:])

    f = None
    for hh in range(4):
        d = jnp.dot(h[:, hh, :].astype(jnp.bfloat16), wf1_ref[hh],
                    preferred_element_type=f32)
        f = d if f is None else f + d
    hrelu = jnp.maximum(f + fb1_ref[...], 0.0)
    o_ref[...] = jnp.dot(hrelu, fw2_ref[...],
                         preferred_element_type=f32) + fb2_ref[...]


def kernel(x, w1, s1, t1, w2, s2, t2, w3, s3, t3, fw1, fb1, fw2, fb2):
    n = x.shape[0]

    xt = jnp.transpose(x, (0, 2, 3, 1))
    xt = jnp.pad(xt, ((0, 0), (2, 2), (2, 2), (0, 0)))
    xt = xt.reshape(n, 36, 108).astype(jnp.bfloat16)

    T1 = _build_T(w1, *_L1)
    T2 = _build_T(w2, *_L2)
    T3 = _build_T(w3, *_L3)
    sc1, sh1 = jnp.tile(s1, (1, 32)), jnp.tile(t1, (1, 32))
    sc2, sh2 = jnp.tile(s2, (1, 16)), jnp.tile(t2, (1, 16))
    sc3, sh3 = jnp.tile(s3, (1, 8)), jnp.tile(t3, (1, 8))

    f4 = fw1.reshape(4, 4, 64, 256)
    Wf1 = jnp.zeros((4, 7, 64, 256), fw1.dtype).at[:, ::2].set(f4)
    Wf1 = Wf1.reshape(4, 448, 256)
    sel1 = jnp.asarray(_SEL1, jnp.bfloat16)
    sel2 = jnp.asarray(_SEL2, jnp.bfloat16)

    full = lambda shape: pl.BlockSpec(shape, lambda i: tuple(0 for _ in shape))
    in_specs = [
        pl.BlockSpec((B, 36, 108), lambda i: (i, 0, 0)),
        full((5, 108, 512)), full((1, 512)), full((1, 512)), full((496, 320)),
        full((5, 320, 512)), full((1, 512)), full((1, 512)), full((480, 320)),
        full((3, 320, 512)), full((1, 512)), full((1, 512)),
        full((4, 448, 256)), full((1, 256)), full((256, 2)), full((1, 2)),
    ]
    out = pl.pallas_call(
        _cnn_body,
        out_shape=jax.ShapeDtypeStruct((n, 2), jnp.float32),
        grid=(n // B,),
        in_specs=in_specs,
        out_specs=pl.BlockSpec((B, 2), lambda i: (i, 0)),
        scratch_shapes=[
            pltpu.VMEM((B, 20, 320), jnp.bfloat16),
            pltpu.VMEM((B, 10, 320), jnp.float32),
        ],
        compiler_params=pltpu.CompilerParams(dimension_semantics=("parallel",)),
    )(xt, T1, sc1, sh1, sel1, T2, sc2, sh2, sel2,
      T3, sc3, sh3, Wf1, fb1, fw2, fb2)
    return out


# speedup vs baseline: 19.1850x; 1.4393x over previous
import numpy as np

import jax
import jax.numpy as jnp
from jax.experimental import pallas as pl
from jax.experimental.pallas import tpu as pltpu

B = 16

_L1 = (5, 3, 16, 32, 36)
_L2 = (5, 16, 32, 16, 20)
_L3 = (3, 32, 64, 8, 10)


def _build_T(w, K, Cin, Cout, Wout, Wpos):
    w4 = w.reshape(K, K, Cin, Cout).astype(jnp.float32)
    j = np.arange(K)[:, None, None]
    wo = np.arange(Wout)[None, :, None]
    wp = np.arange(Wpos)[None, None, :]
    O = (wp == wo + j).astype(np.float32)
    T = jnp.einsum('jwp,ijcd->ipcwd', O, w4)
    return T.reshape(K, Wpos * Cin, Wout * Cout).astype(jnp.bfloat16)


def _rowsel(H, parity):
    S = np.zeros((B * H // 2, B * H), np.float32)
    for b in range(B):
        for q in range(H // 2):
            S[b * (H // 2) + q, b * H + 2 * q + parity] = 1.0
    return S


def _laneshift(W, C):
    R = np.zeros((W * C, W * C), np.float32)
    for w in range(W - 1):
        for c in range(C):
            R[(w + 1) * C + c, w * C + c] = 1.0
    return R


def _build_sel(Wout_pooled, C, halo, in_lanes, out_lanes):
    S = np.zeros((in_lanes, out_lanes), np.float32)
    for w in range(Wout_pooled):
        for c in range(C):
            S[(2 * w) * C + c, (halo + w) * C + c] = 1.0
    return S


_SE1, _SO1 = _rowsel(32, 0), _rowsel(32, 1)
_SE2, _SO2 = _rowsel(16, 0), _rowsel(16, 1)
_SE3, _SO3 = _rowsel(8, 0), _rowsel(8, 1)
_R1 = _laneshift(32, 16)
_R2 = _laneshift(16, 32)
_R3 = _laneshift(8, 64)
_SEL1 = _build_sel(16, 16, 2, 512, 320)
_SEL2 = _build_sel(8, 32, 1, 512, 320)


def _pool(y, se_ref, so_ref, r_ref):
    f32 = jnp.float32
    yb = y.astype(jnp.bfloat16)
    mv = jnp.maximum(jnp.dot(se_ref[...], yb, preferred_element_type=f32),
                     jnp.dot(so_ref[...], yb, preferred_element_type=f32))
    mr = jnp.dot(mv.astype(jnp.bfloat16), r_ref[...],
                 preferred_element_type=f32)
    return jnp.maximum(mv, mr).astype(jnp.bfloat16)


def _cnn_body(xt_ref, t1_ref, sc1_ref, sh1_ref,
              se1_ref, so1_ref, r1_ref, sel1_ref,
              t2_ref, sc2_ref, sh2_ref,
              se2_ref, so2_ref, r2_ref, sel2_ref,
              t3_ref, sc3_ref, sh3_ref,
              se3_ref, so3_ref, r3_ref,
              wf1_ref, fb1_ref, fw2_ref, fb2_ref,
              o_ref, a2_ref, a3_ref):
    f32 = jnp.float32

    acc = None
    for i in range(5):
        lhs = xt_ref[:, i:i + 32, :].reshape(B * 32, 108)
        d = jnp.dot(lhs, t1_ref[i], preferred_element_type=f32)
        acc = d if acc is None else acc + d
    y = jnp.maximum(acc * sc1_ref[...] + sh1_ref[...], 0.0)
    h = _pool(y, se1_ref, so1_ref, r1_ref)
    p = jnp.dot(h, sel1_ref[...], preferred_element_type=f32)
    a2_ref[:, 0:2, :] = jnp.zeros((B, 2, 320), jnp.bfloat16)
    a2_ref[:, 18:20, :] = jnp.zeros((B, 2, 320), jnp.bfloat16)
    a2_ref[:, 2:18, :] = p.astype(jnp.bfloat16).reshape(B, 16, 320)

    acc = None
    for i in range(5):
        lhs = a2_ref[:, i:i + 16, :].reshape(B * 16, 320)
        d = jnp.dot(lhs, t2_ref[i], preferred_element_type=f32)
        acc = d if acc is None else acc + d
    y = jnp.maximum(acc * sc2_ref[...] + sh2_ref[...], 0.0)
    h = _pool(y, se2_ref, so2_ref, r2_ref)
    p = jnp.dot(h, sel2_ref[...], preferred_element_type=f32)
    a3_ref[:, 0:1, :] = jnp.zeros((B, 1, 320), f32)
    a3_ref[:, 9:10, :] = jnp.zeros((B, 1, 320), f32)
    a3_ref[:, 1:9, :] = p.reshape(B, 8, 320)

    acc = None
    for i in range(3):
        lhs = a3_ref[:, i:i + 8, :].reshape(B * 8, 320).astype(jnp.bfloat16)
        d = jnp.dot(lhs, t3_ref[i], preferred_element_type=f32)
        acc = d if acc is None else acc + d
    y = jnp.maximum(acc * sc3_ref[...] + sh3_ref[...], 0.0)
    h = _pool(y, se3_ref, so3_ref, r3_ref)

    h = h.reshape(B, 4, 512)
    f = None
    for hh in range(4):
        d = jnp.dot(h[:, hh, :], wf1_ref[hh],
                    preferred_element_type=f32)
        f = d if f is None else f + d
    hrelu = jnp.maximum(f + fb1_ref[...], 0.0)
    o_ref[...] = jnp.dot(hrelu, fw2_ref[...],
                         preferred_element_type=f32) + fb2_ref[...]


def kernel(x, w1, s1, t1, w2, s2, t2, w3, s3, t3, fw1, fb1, fw2, fb2):
    n = x.shape[0]

    xt = jnp.transpose(x, (0, 2, 3, 1))
    xt = jnp.pad(xt, ((0, 0), (2, 2), (2, 2), (0, 0)))
    xt = xt.reshape(n, 36, 108).astype(jnp.bfloat16)

    T1 = _build_T(w1, *_L1)
    T2 = _build_T(w2, *_L2)
    T3 = _build_T(w3, *_L3)
    sc1, sh1 = jnp.tile(s1, (1, 32)), jnp.tile(t1, (1, 32))
    sc2, sh2 = jnp.tile(s2, (1, 16)), jnp.tile(t2, (1, 16))
    sc3, sh3 = jnp.tile(s3, (1, 8)), jnp.tile(t3, (1, 8))

    f4 = fw1.reshape(4, 4, 64, 256)
    Wf1 = jnp.zeros((4, 8, 64, 256), fw1.dtype).at[:, 0::2].set(f4)
    Wf1 = Wf1.reshape(4, 512, 256)

    bf = jnp.bfloat16
    se1, so1, r1 = jnp.asarray(_SE1, bf), jnp.asarray(_SO1, bf), jnp.asarray(_R1, bf)
    se2, so2, r2 = jnp.asarray(_SE2, bf), jnp.asarray(_SO2, bf), jnp.asarray(_R2, bf)
    se3, so3, r3 = jnp.asarray(_SE3, bf), jnp.asarray(_SO3, bf), jnp.asarray(_R3, bf)
    sel1, sel2 = jnp.asarray(_SEL1, bf), jnp.asarray(_SEL2, bf)

    full = lambda shape: pl.BlockSpec(shape, lambda i: tuple(0 for _ in shape))
    in_specs = [
        pl.BlockSpec((B, 36, 108), lambda i: (i, 0, 0)),
        full((5, 108, 512)), full((1, 512)), full((1, 512)),
        full((256, 512)), full((256, 512)), full((512, 512)), full((512, 320)),
        full((5, 320, 512)), full((1, 512)), full((1, 512)),
        full((128, 256)), full((128, 256)), full((512, 512)), full((512, 320)),
        full((3, 320, 512)), full((1, 512)), full((1, 512)),
        full((64, 128)), full((64, 128)), full((512, 512)),
        full((4, 512, 256)), full((1, 256)), full((256, 2)), full((1, 2)),
    ]
    out = pl.pallas_call(
        _cnn_body,
        out_shape=jax.ShapeDtypeStruct((n, 2), jnp.float32),
        grid=(n // B,),
        in_specs=in_specs,
        out_specs=pl.BlockSpec((B, 2), lambda i: (i, 0)),
        scratch_shapes=[
            pltpu.VMEM((B, 20, 320), jnp.bfloat16),
            pltpu.VMEM((B, 10, 320), jnp.float32),
        ],
        compiler_params=pltpu.CompilerParams(
            dimension_semantics=("arbitrary",)),
    )(xt, T1, sc1, sh1, se1, so1, r1, sel1,
      T2, sc2, sh2, se2, so2, r2, sel2,
      T3, sc3, sh3, se3, so3, r3,
      Wf1, fb1, fw2, fb2)
    return out


# speedup vs baseline: 22.8753x; 1.1924x over previous
import numpy as np

import jax
import jax.numpy as jnp
from jax.experimental import pallas as pl
from jax.experimental.pallas import tpu as pltpu

B = 32
G = 16
NG = B // G

_L1 = (5, 3, 16, 32, 36)
_L2 = (5, 16, 32, 16, 20)
_L3 = (3, 32, 64, 8, 10)


def _build_T(w, K, Cin, Cout, Wout, Wpos):
    w4 = w.reshape(K, K, Cin, Cout).astype(jnp.float32)
    j = np.arange(K)[:, None, None]
    wo = np.arange(Wout)[None, :, None]
    wp = np.arange(Wpos)[None, None, :]
    O = (wp == wo + j).astype(np.float32)
    T = jnp.einsum('jwp,ijcd->ipcwd', O, w4)
    return T.reshape(K, Wpos * Cin, Wout * Cout).astype(jnp.bfloat16)


def _rowsel(H, parity):
    S = np.zeros((G * H // 2, G * H), np.float32)
    for b in range(G):
        for q in range(H // 2):
            S[b * (H // 2) + q, b * H + 2 * q + parity] = 1.0
    return S


def _laneshift(W, C):
    R = np.zeros((W * C, W * C), np.float32)
    for w in range(W - 1):
        for c in range(C):
            R[(w + 1) * C + c, w * C + c] = 1.0
    return R


def _build_sel(Wout_pooled, C, halo, in_lanes, out_lanes):
    S = np.zeros((in_lanes, out_lanes), np.float32)
    for w in range(Wout_pooled):
        for c in range(C):
            S[(2 * w) * C + c, (halo + w) * C + c] = 1.0
    return S


_SE1, _SO1 = _rowsel(32, 0), _rowsel(32, 1)
_SE2, _SO2 = _rowsel(16, 0), _rowsel(16, 1)
_SE3, _SO3 = _rowsel(8, 0), _rowsel(8, 1)
_R1 = _laneshift(32, 16)
_R2 = _laneshift(16, 32)
_R3 = _laneshift(8, 64)
_SEL1 = _build_sel(16, 16, 2, 512, 320)
_SEL2 = _build_sel(8, 32, 1, 512, 320)


def _pool(y, se_ref, so_ref, r_ref):
    f32 = jnp.float32
    yb = y.astype(jnp.bfloat16)
    mv = jnp.maximum(jnp.dot(se_ref[...], yb, preferred_element_type=f32),
                     jnp.dot(so_ref[...], yb, preferred_element_type=f32))
    mr = jnp.dot(mv.astype(jnp.bfloat16), r_ref[...],
                 preferred_element_type=f32)
    return jnp.maximum(mv, mr).astype(jnp.bfloat16)


def _cnn_body(xt_ref, t1_ref, sc1_ref, sh1_ref,
              se1_ref, so1_ref, r1_ref, sel1_ref,
              t2_ref, sc2_ref, sh2_ref,
              se2_ref, so2_ref, r2_ref, sel2_ref,
              t3_ref, sc3_ref, sh3_ref,
              se3_ref, so3_ref, r3_ref,
              wf1_ref, fb1_ref, fw2_ref, fb2_ref,
              o_ref, a2_ref, a3_ref):
    f32 = jnp.float32

    acc = None
    for i in range(5):
        lhs = xt_ref[:, i:i + 32, :].reshape(B * 32, 108)
        d = jnp.dot(lhs, t1_ref[i], preferred_element_type=f32)
        acc = d if acc is None else acc + d
    y = jnp.maximum(acc * sc1_ref[...] + sh1_ref[...], 0.0)
    a2_ref[:, 0:2, :] = jnp.zeros((B, 2, 320), jnp.bfloat16)
    a2_ref[:, 18:20, :] = jnp.zeros((B, 2, 320), jnp.bfloat16)
    for g in range(NG):
        h = _pool(y[g * G * 32:(g + 1) * G * 32],
                  se1_ref, so1_ref, r1_ref)
        p = jnp.dot(h, sel1_ref[...], preferred_element_type=f32)
        a2_ref[g * G:(g + 1) * G, 2:18, :] = (
            p.astype(jnp.bfloat16).reshape(G, 16, 320))

    acc = None
    for i in range(5):
        lhs = a2_ref[:, i:i + 16, :].reshape(B * 16, 320)
        d = jnp.dot(lhs, t2_ref[i], preferred_element_type=f32)
        acc = d if acc is None else acc + d
    y = jnp.maximum(acc * sc2_ref[...] + sh2_ref[...], 0.0)
    a3_ref[:, 0:1, :] = jnp.zeros((B, 1, 320), f32)
    a3_ref[:, 9:10, :] = jnp.zeros((B, 1, 320), f32)
    for g in range(NG):
        h = _pool(y[g * G * 16:(g + 1) * G * 16],
                  se2_ref, so2_ref, r2_ref)
        p = jnp.dot(h, sel2_ref[...], preferred_element_type=f32)
        a3_ref[g * G:(g + 1) * G, 1:9, :] = p.reshape(G, 8, 320)

    acc = None
    for i in range(3):
        lhs = a3_ref[:, i:i + 8, :].reshape(B * 8, 320).astype(jnp.bfloat16)
        d = jnp.dot(lhs, t3_ref[i], preferred_element_type=f32)
        acc = d if acc is None else acc + d
    y = jnp.maximum(acc * sc3_ref[...] + sh3_ref[...], 0.0)
    hs = [_pool(y[g * G * 8:(g + 1) * G * 8],
                se3_ref, so3_ref, r3_ref) for g in range(NG)]
    h = hs[0] if NG == 1 else jnp.concatenate(hs, axis=0)

    h = h.reshape(B, 4, 512)
    f = None
    for hh in range(4):
        d = jnp.dot(h[:, hh, :], wf1_ref[hh],
                    preferred_element_type=f32)
        f = d if f is None else f + d
    hrelu = jnp.maximum(f + fb1_ref[...], 0.0)
    o_ref[...] = jnp.dot(hrelu, fw2_ref[...],
                         preferred_element_type=f32) + fb2_ref[...]


def kernel(x, w1, s1, t1, w2, s2, t2, w3, s3, t3, fw1, fb1, fw2, fb2):
    n = x.shape[0]

    xt = jnp.transpose(x, (0, 2, 3, 1))
    xt = jnp.pad(xt, ((0, 0), (2, 2), (2, 2), (0, 0)))
    xt = xt.reshape(n, 36, 108).astype(jnp.bfloat16)

    T1 = _build_T(w1, *_L1)
    T2 = _build_T(w2, *_L2)
    T3 = _build_T(w3, *_L3)
    sc1, sh1 = jnp.tile(s1, (1, 32)), jnp.tile(t1, (1, 32))
    sc2, sh2 = jnp.tile(s2, (1, 16)), jnp.tile(t2, (1, 16))
    sc3, sh3 = jnp.tile(s3, (1, 8)), jnp.tile(t3, (1, 8))

    f4 = fw1.reshape(4, 4, 64, 256)
    Wf1 = jnp.zeros((4, 8, 64, 256), fw1.dtype).at[:, 0::2].set(f4)
    Wf1 = Wf1.reshape(4, 512, 256)

    bf = jnp.bfloat16
    se1, so1, r1 = jnp.asarray(_SE1, bf), jnp.asarray(_SO1, bf), jnp.asarray(_R1, bf)
    se2, so2, r2 = jnp.asarray(_SE2, bf), jnp.asarray(_SO2, bf), jnp.asarray(_R2, bf)
    se3, so3, r3 = jnp.asarray(_SE3, bf), jnp.asarray(_SO3, bf), jnp.asarray(_R3, bf)
    sel1, sel2 = jnp.asarray(_SEL1, bf), jnp.asarray(_SEL2, bf)

    full = lambda shape: pl.BlockSpec(shape, lambda i: tuple(0 for _ in shape))
    in_specs = [
        pl.BlockSpec((B, 36, 108), lambda i: (i, 0, 0)),
        full((5, 108, 512)), full((1, 512)), full((1, 512)),
        full((256, 512)), full((256, 512)), full((512, 512)), full((512, 320)),
        full((5, 320, 512)), full((1, 512)), full((1, 512)),
        full((128, 256)), full((128, 256)), full((512, 512)), full((512, 320)),
        full((3, 320, 512)), full((1, 512)), full((1, 512)),
        full((64, 128)), full((64, 128)), full((512, 512)),
        full((4, 512, 256)), full((1, 256)), full((256, 2)), full((1, 2)),
    ]
    out = pl.pallas_call(
        _cnn_body,
        out_shape=jax.ShapeDtypeStruct((n, 2), jnp.float32),
        grid=(n // B,),
        in_specs=in_specs,
        out_specs=pl.BlockSpec((B, 2), lambda i: (i, 0)),
        scratch_shapes=[
            pltpu.VMEM((B, 20, 320), jnp.bfloat16),
            pltpu.VMEM((B, 10, 320), jnp.float32),
        ],
        compiler_params=pltpu.CompilerParams(
            dimension_semantics=("arbitrary",)),
    )(xt, T1, sc1, sh1, se1, so1, r1, sel1,
      T2, sc2, sh2, se2, so2, r2, sel2,
      T3, sc3, sh3, se3, so3, r3,
      Wf1, fb1, fw2, fb2)
    return out
